# baseline (speedup 1.0000x reference)
import functools

import jax
import jax.numpy as jnp
from jax import lax
from jax.experimental import pallas as pl
from jax.experimental.pallas import tpu as pltpu
from jax.experimental.pallas import tpu_sc as plsc

D = 64
L = 16
SCALE = 8.0
NC, NS = 2, 16
NW = NC * NS
CHUNK = 128


def _emb_call(idx, table, tot, n_chunks):
    mesh = plsc.VectorSubcoreMesh(core_axis_name="c", subcore_axis_name="s")

    @functools.partial(
        pl.kernel,
        mesh=mesh,
        out_type=jax.ShapeDtypeStruct((tot, D), jnp.float32),
        scratch_types=[
            pltpu.VMEM((n_chunks, CHUNK), jnp.int32),
            pltpu.VMEM((CHUNK, D), jnp.float32),
            pltpu.SemaphoreType.DMA,
        ],
        compiler_params=pltpu.CompilerParams(use_tc_tiling_on_sc=False),
    )
    def emb(idx_hbm, table_hbm, out_hbm, idx_v, rows_v, sem):
        wid = lax.axis_index("s") * NC + lax.axis_index("c")
        pltpu.sync_copy(idx_hbm.at[pl.ds(wid * n_chunks, n_chunks)], idx_v)

        def chunk_body(g, carry):
            pltpu.async_copy(table_hbm.at[idx_v.at[g]], rows_v, sem).wait()

            def scale_row(r, c):
                for j in range(D // L):
                    rows_v[r, pl.ds(j * L, L)] = rows_v[r, pl.ds(j * L, L)] * SCALE
                return c

            lax.fori_loop(0, CHUNK, scale_row, 0)
            pltpu.sync_copy(
                rows_v, out_hbm.at[pl.ds((wid * n_chunks + g) * CHUNK, CHUNK)]
            )
            return carry

        lax.fori_loop(0, n_chunks, chunk_body, 0)

    return emb(idx, table)


def kernel(x, table):
    batch, seq = x.shape
    tot = batch * seq
    n_chunks = tot // (NW * CHUNK)
    idx = x.reshape(NW * n_chunks, CHUNK)
    out = _emb_call(idx, table, tot, n_chunks)
    return out.reshape(batch, seq, D)


# speedup vs baseline: 1.2106x; 1.2106x over previous
import functools

import jax
import jax.numpy as jnp
from jax import lax
from jax.experimental import pallas as pl
from jax.experimental.pallas import tpu as pltpu
from jax.experimental.pallas import tpu_sc as plsc

D = 64
L = 16
SCALE = 8.0
NC, NS = 2, 16
NW = NC * NS
CHUNK = 128
NG = 4
NSB = 4


def _emb_call(idx, table, tot, n_chunks):
    mesh = plsc.VectorSubcoreMesh(core_axis_name="c", subcore_axis_name="s")
    n_outer = n_chunks // NG

    @functools.partial(
        pl.kernel,
        mesh=mesh,
        out_type=jax.ShapeDtypeStruct((tot, D), jnp.float32),
        scratch_types=[
            pltpu.VMEM((n_chunks, CHUNK), jnp.int32),
            pltpu.VMEM((NG, CHUNK, D), jnp.float32),
            pltpu.VMEM((NSB, CHUNK, D), jnp.float32),
        ]
        + [pltpu.SemaphoreType.DMA] * (NG + NSB),
        compiler_params=pltpu.CompilerParams(use_tc_tiling_on_sc=False),
    )
    def emb(idx_hbm, table_hbm, out_hbm, idx_v, gbuf, sbuf, *sems):
        gsems, ssems = sems[:NG], sems[NG:]
        wid = lax.axis_index("s") * NC + lax.axis_index("c")
        base = wid * n_chunks
        pltpu.sync_copy(idx_hbm.at[pl.ds(base, n_chunks)], idx_v)

        for b in range(NG):
            pltpu.async_copy(table_hbm.at[idx_v.at[b]], gbuf.at[b], gsems[b])

        def outer(o, carry):
            for b in range(NG):
                g = o * NG + b
                pltpu.make_async_copy(
                    table_hbm.at[idx_v.at[g]], gbuf.at[b], gsems[b]
                ).wait()

                @pl.when(o > 0)
                def _():
                    pltpu.make_async_copy(
                        sbuf.at[b],
                        out_hbm.at[pl.ds((base + g - NSB) * CHUNK, CHUNK)],
                        ssems[b],
                    ).wait()

                @plsc.parallel_loop(0, CHUNK, step=1, unroll=8)
                def _(r):
                    for j in range(D // L):
                        sl = pl.ds(j * L, L)
                        sbuf[b, r, sl] = gbuf[b, r, sl] * SCALE

                pltpu.async_copy(
                    sbuf.at[b],
                    out_hbm.at[pl.ds((base + g) * CHUNK, CHUNK)],
                    ssems[b],
                )

                @pl.when(o < n_outer - 1)
                def _():
                    pltpu.async_copy(
                        table_hbm.at[idx_v.at[g + NG]], gbuf.at[b], gsems[b]
                    )

            return carry

        lax.fori_loop(0, n_outer, outer, 0)

        for b in range(NSB):
            g = (n_outer - 1) * NG + b
            pltpu.make_async_copy(
                sbuf.at[b],
                out_hbm.at[pl.ds((base + g) * CHUNK, CHUNK)],
                ssems[b],
            ).wait()

    return emb(idx, table)


def kernel(x, table):
    batch, seq = x.shape
    tot = batch * seq
    n_chunks = tot // (NW * CHUNK)
    idx = x.reshape(NW * n_chunks, CHUNK)
    out = _emb_call(idx, table, tot, n_chunks)
    return out.reshape(batch, seq, D)


# speedup vs baseline: 1.5341x; 1.2672x over previous
import functools

import jax
import jax.numpy as jnp
from jax import lax
from jax.experimental import pallas as pl
from jax.experimental.pallas import tpu as pltpu
from jax.experimental.pallas import tpu_sc as plsc

D = 64
L = 16
SCALE = 8.0
NC, NS = 2, 16
NW = NC * NS
CHUNK = 128
NBUF = 8
LOOKAHEAD = 4

VBLK = 2048
NPAIR = 245
PACKED_ROWS = NPAIR * VBLK


def _table_transpose_scale(table_t):
    def body(i_ref, o_ref):
        o_ref[:, 0:D] = i_ref[:, 0:VBLK].T * SCALE
        o_ref[:, D : 2 * D] = i_ref[:, VBLK : 2 * VBLK].T * SCALE

    return pl.pallas_call(
        body,
        grid=(NPAIR,),
        in_specs=[pl.BlockSpec((D, 2 * VBLK), lambda j: (0, j))],
        out_specs=pl.BlockSpec((VBLK, 2 * D), lambda j: (j, 0)),
        out_shape=jax.ShapeDtypeStruct((PACKED_ROWS, 2 * D), jnp.float32),
    )(table_t)


def _emb_call(idx, table_rm, tot, n_chunks):
    mesh = plsc.VectorSubcoreMesh(core_axis_name="c", subcore_axis_name="s")

    @functools.partial(
        pl.kernel,
        mesh=mesh,
        out_type=jax.ShapeDtypeStruct((tot, D), jnp.float32),
        scratch_types=[
            pltpu.VMEM((n_chunks, CHUNK), jnp.int32),
            pltpu.VMEM((NBUF, CHUNK, D), jnp.float32),
        ]
        + [pltpu.SemaphoreType.DMA] * (2 * NBUF),
        compiler_params=pltpu.CompilerParams(use_tc_tiling_on_sc=False),
    )
    def emb(idx_hbm, table_hbm, out_hbm, idx_v, gbuf, *sems):
        gsems, ssems = sems[:NBUF], sems[NBUF:]
        wid = lax.axis_index("s") * NC + lax.axis_index("c")
        base = wid * n_chunks
        pltpu.sync_copy(idx_hbm.at[pl.ds(base, n_chunks)], idx_v)

        @plsc.parallel_loop(0, n_chunks, step=1, unroll=2)
        def _(r):
            for j in range(CHUNK // L):
                sl = pl.ds(j * L, L)
                v = idx_v[r, sl]
                k = jax.lax.shift_right_logical(v, 12)
                t = jax.lax.bitwise_and(v, VBLK - 1)
                h = jax.lax.bitwise_and(jax.lax.shift_right_logical(v, 11), 1)
                idx_v[r, sl] = (
                    jax.lax.shift_left(k, 12) + jax.lax.shift_left(t, 1) + h
                )

        def gather(g, b):
            pltpu.async_copy(table_hbm.at[idx_v.at[g]], gbuf.at[b], gsems[b])

        def store(g, b, wait):
            cp = pltpu.make_async_copy(
                gbuf.at[b], out_hbm.at[pl.ds((base + g) * CHUNK, CHUNK)], ssems[b]
            )
            cp.wait() if wait else cp.start()

        for g in range(LOOKAHEAD):
            gather(g, g)

        n_outer = n_chunks // NBUF

        def outer(o, carry):
            for b in range(NBUF):
                g = o * NBUF + b
                pltpu.make_async_copy(
                    table_hbm.at[idx_v.at[g]], gbuf.at[b], gsems[b]
                ).wait()
                store(g, b, wait=False)
                h = g + LOOKAHEAD
                bh = (b + LOOKAHEAD) % NBUF

                def _wait_prev_store():
                    store(g - LOOKAHEAD, bh, wait=True)

                if b >= LOOKAHEAD:
                    _wait_prev_store()
                else:
                    pl.when(o > 0)(_wait_prev_store)

                @pl.when(h < n_chunks)
                def _():
                    gather(h, bh)

            return carry

        lax.fori_loop(0, n_outer, outer, 0)

        for k in range(LOOKAHEAD):
            g = n_chunks - LOOKAHEAD + k
            store(g, g % NBUF, wait=True)

    return emb(idx, table_rm)


def kernel(x, table):
    batch, seq = x.shape
    tot = batch * seq
    n_chunks = tot // (NW * CHUNK)
    idx = x.reshape(NW * n_chunks, CHUNK)
    table_rm = _table_transpose_scale(table.T).reshape(2 * PACKED_ROWS, D)
    out = _emb_call(idx, table_rm, tot, n_chunks)
    return out.reshape(batch, seq, D)
